# Initial kernel scaffold; baseline (speedup 1.0000x reference)
import jax
import jax.numpy as jnp
from jax.experimental import pallas as pl


def kernel(x, W_rel1, b_rel1, W_root1, W_rel2, b_rel2, W_root2, gamma, beta):
    raise NotImplementedError("write your pallas kernel here")


# speedup vs baseline: 8.8008x; 8.8008x over previous
import functools

import jax
import jax.numpy as jnp
from jax.experimental import pallas as pl
from jax.experimental.pallas import tpu as pltpu

_N = 6
_C = 64
_TB = 1024


def _gelu_exact(v):
    return 0.5 * v * (1.0 + jax.lax.erf(v * 0.7071067811865476))


def _body(x_ref, w1_ref, b1_ref, w2_ref, b2_ref, gs_ref, bt_ref, o_ref):
    xs = [x_ref[i] for i in range(_N)]
    s = xs[0] + xs[1] + xs[2] + xs[3] + xs[4] + xs[5]
    p25 = xs[2] + xs[5]
    p34 = xs[3] + xs[4]
    aggs = [s - xs[0], s - xs[1], s - p25, s - p34, s - p34, s - p25]

    w1 = w1_ref[...]
    b1 = b1_ref[...]
    hs = []
    for i in range(_N):
        cat = jnp.concatenate([aggs[i], xs[i]], axis=-1)
        h = jnp.dot(cat, w1, preferred_element_type=jnp.float32) + b1
        hs.append(_gelu_exact(h))

    s2 = hs[0] + hs[1] + hs[2] + hs[3] + hs[4] + hs[5]
    q25 = hs[2] + hs[5]
    q34 = hs[3] + hs[4]
    agg2 = [s2 - hs[0], s2 - hs[1], s2 - q25, s2 - q34, s2 - q34, s2 - q25]

    w2 = w2_ref[...]
    b2 = b2_ref[...]
    acc = jnp.zeros_like(hs[0])
    for i in range(_N):
        cat = jnp.concatenate([agg2[i], hs[i]], axis=-1)
        h2 = jnp.dot(cat, w2, preferred_element_type=jnp.float32) + b2
        acc = acc + _gelu_exact(h2)

    o_ref[...] = acc * gs_ref[...] + bt_ref[...]


@functools.partial(jax.jit, static_argnames=())
def kernel(x, W_rel1, b_rel1, W_root1, W_rel2, b_rel2, W_root2, gamma, beta):
    n, b, c = x.shape
    w1 = jnp.concatenate([W_rel1.T, W_root1.T], axis=0)
    w2 = jnp.concatenate([W_rel2.T, W_root2.T], axis=0)
    b1 = b_rel1.reshape(1, c)
    b2 = b_rel2.reshape(1, c)
    gs = (gamma / (n * jnp.sqrt(1.0 + 1e-5))).reshape(1, c)
    bt = beta.reshape(1, c)

    grid = (b // _TB,)
    return pl.pallas_call(
        _body,
        grid=grid,
        in_specs=[
            pl.BlockSpec((n, _TB, c), lambda i: (0, i, 0)),
            pl.BlockSpec((2 * c, c), lambda i: (0, 0)),
            pl.BlockSpec((1, c), lambda i: (0, 0)),
            pl.BlockSpec((2 * c, c), lambda i: (0, 0)),
            pl.BlockSpec((1, c), lambda i: (0, 0)),
            pl.BlockSpec((1, c), lambda i: (0, 0)),
            pl.BlockSpec((1, c), lambda i: (0, 0)),
        ],
        out_specs=pl.BlockSpec((_TB, c), lambda i: (i, 0)),
        out_shape=jax.ShapeDtypeStruct((b, c), jnp.float32),
        compiler_params=pltpu.CompilerParams(
            dimension_semantics=("parallel",),
        ),
    )(x, w1, b1, w2, b2, gs, bt)
